# Initial kernel scaffold; baseline (speedup 1.0000x reference)
import functools

import jax
import jax.numpy as jnp
from jax import lax
from jax.experimental import pallas as pl
from jax.experimental.pallas import tpu as pltpu
from jax.experimental.pallas import tpu_sc as plsc

N = 10000
E = 320000
D = 128
NC, NS, L = 2, 16, 16
NW = NC * NS
EPT = 10240
CH = 128
NCH = EPT // CH
E_PAD = NW * EPT
N_ACC = 10112
STRIPE = N_ACC // NS
DEG_R = 80
BLK = 632


def _sc_agg_body(compute_deg, *refs):
    if compute_deg:
        (h, src, dst, part_out, deg_out,
         src_v, dst_v, rows_v, zero_v, deg_v, idx_v, acc_s, dega_s, gsem) = refs
    else:
        (h, src, dst, part_out,
         src_v, dst_v, rows_v, zero_v, acc_s, gsem) = refs

    c = lax.axis_index("c")
    s = lax.axis_index("s")
    wid = c * NS + s

    def _zrow(i, _):
        for k in range(D // L):
            zero_v[i, pl.ds(k * L, L)] = jnp.zeros((L,), jnp.float32)
        return 0
    lax.fori_loop(0, CH, _zrow, 0)

    base = s * STRIPE
    off = 0
    for sz in (128, 128, 128, 128, STRIPE - 4 * 128):
        pltpu.sync_copy(zero_v.at[pl.ds(0, sz)], acc_s.at[pl.ds(base + off, sz)])
        off += sz

    if compute_deg:
        def _zdeg(i, _):
            for k in range(D // L):
                deg_v[i, pl.ds(k * L, L)] = jnp.zeros((L,), jnp.float32)
            return 0
        lax.fori_loop(0, DEG_R, _zdeg, 0)

        def _ziota(i, _):
            idx_v[pl.ds(i * L, L)] = lax.iota(jnp.int32, (L,)) + i * L
            return 0
        lax.fori_loop(0, DEG_R // L, _ziota, 0)

        @pl.when(s == 0)
        def _():
            pltpu.sync_copy(zero_v.at[pl.ds(0, DEG_R)], dega_s)

    pltpu.sync_copy(src.at[wid], src_v)
    pltpu.sync_copy(dst.at[wid], dst_v)

    plsc.subcore_barrier()

    ones = jnp.ones((L,), jnp.float32)

    pltpu.async_copy(h.at[src_v.at[0]], rows_v.at[0], gsem)

    def _chunks(jo, _):
        for b in range(2):
            j = jo * 2 + b
            pltpu.make_async_copy(h.at[src_v.at[j]], rows_v.at[b], gsem).wait()

            @pl.when(j + 1 < NCH)
            def _():
                pltpu.async_copy(h.at[src_v.at[j + 1]], rows_v.at[1 - b], gsem)

            pltpu.sync_copy(rows_v.at[b], acc_s.at[dst_v.at[j]], add=True)

            if compute_deg:
                for k in range(CH // L):
                    idx16 = dst_v[j, pl.ds(k * L, L)]
                    row16 = lax.shift_right_logical(idx16, 7)
                    col16 = lax.bitwise_and(idx16, 127)
                    plsc.addupdate_scatter(deg_v, [row16, col16], ones)
        return 0
    lax.fori_loop(0, NCH // 2, _chunks, 0)

    if compute_deg:
        pltpu.sync_copy(deg_v, dega_s.at[idx_v], add=True)

    plsc.subcore_barrier()

    pltpu.sync_copy(acc_s.at[pl.ds(base, STRIPE)],
                    part_out.at[c, pl.ds(base, STRIPE)])
    if compute_deg:
        r = DEG_R // NS
        pltpu.sync_copy(dega_s.at[pl.ds(s * r, r)],
                        deg_out.at[c, pl.ds(s * r, r)])


def _make_sc_agg(compute_deg):
    mesh = plsc.VectorSubcoreMesh(core_axis_name="c", subcore_axis_name="s")
    out_type = [jax.ShapeDtypeStruct((NC, N_ACC, D), jnp.float32)]
    scratch = [
        pltpu.VMEM((NCH, CH), jnp.int32),
        pltpu.VMEM((NCH, CH), jnp.int32),
        pltpu.VMEM((2, CH, D), jnp.float32),
        pltpu.VMEM((CH, D), jnp.float32),
    ]
    if compute_deg:
        out_type.append(jax.ShapeDtypeStruct((NC, DEG_R, D), jnp.float32))
        scratch += [
            pltpu.VMEM((DEG_R, D), jnp.float32),
            pltpu.VMEM((DEG_R,), jnp.int32),
        ]
    scratch.append(pltpu.VMEM_SHARED((N_ACC, D), jnp.float32))
    if compute_deg:
        scratch.append(pltpu.VMEM_SHARED((DEG_R, D), jnp.float32))
    scratch.append(pltpu.SemaphoreType.DMA)

    return pl.kernel(
        functools.partial(_sc_agg_body, compute_deg),
        out_type=tuple(out_type),
        mesh=mesh,
        scratch_types=tuple(scratch),
    )


def _dense_body(relu, h_ref, p_ref, d0_ref, d1_ref, ws_ref, wn_ref, b_ref, o_ref):
    deg = d0_ref[...] + d1_ref[...]
    recip = 1.0 / jnp.maximum(deg, 1.0)
    ssum = p_ref[0] + p_ref[1]
    hn = ssum * recip
    z = (jnp.dot(h_ref[...], ws_ref[...], preferred_element_type=jnp.float32)
         + jnp.dot(hn, wn_ref[...], preferred_element_type=jnp.float32)
         + b_ref[...])
    o_ref[...] = jnp.maximum(z, 0.0) if relu else z


def _dense(h, part, d0, d1, Ws, Wn, b, relu):
    return pl.pallas_call(
        functools.partial(_dense_body, relu),
        grid=(NS,),
        in_specs=[
            pl.BlockSpec((BLK, D), lambda i: (i, 0)),
            pl.BlockSpec((NC, BLK, D), lambda i: (0, i, 0)),
            pl.BlockSpec((BLK, 1), lambda i: (i, 0)),
            pl.BlockSpec((BLK, 1), lambda i: (i, 0)),
            pl.BlockSpec((D, D), lambda i: (0, 0)),
            pl.BlockSpec((D, D), lambda i: (0, 0)),
            pl.BlockSpec((1, D), lambda i: (0, 0)),
        ],
        out_specs=pl.BlockSpec((BLK, D), lambda i: (i, 0)),
        out_shape=jax.ShapeDtypeStruct((N, D), jnp.float32),
    )(h, part, d0, d1, Ws, Wn, b)


_sc_agg_deg = _make_sc_agg(True)
_sc_agg = _make_sc_agg(False)


def kernel(x, edge_index, W_self1, W_neigh1, b1, W_self2, W_neigh2, b2):
    src = edge_index[0]
    dst = edge_index[1]
    src_p = jnp.concatenate(
        [src, jnp.zeros((E_PAD - E,), jnp.int32)]).reshape(NW, NCH, CH)
    dst_p = jnp.concatenate(
        [dst, jnp.full((E_PAD - E,), N_ACC - 1, jnp.int32)]).reshape(NW, NCH, CH)

    part1, degp = _sc_agg_deg(x, src_p, dst_p)
    d0 = degp[0].reshape(-1)[:N_ACC, None]
    d1 = degp[1].reshape(-1)[:N_ACC, None]
    b1r = b1.reshape(1, D)
    b2r = b2.reshape(1, D)

    h1 = _dense(x, part1, d0, d1, W_self1, W_neigh1, b1r, relu=True)
    (part2,) = _sc_agg(h1, src_p, dst_p)
    out = _dense(h1, part2, d0, d1, W_self2, W_neigh2, b2r, relu=False)
    return out


# speedup vs baseline: 3.6464x; 3.6464x over previous
import functools

import jax
import jax.numpy as jnp
from jax import lax
from jax.experimental import pallas as pl
from jax.experimental.pallas import tpu as pltpu
from jax.experimental.pallas import tpu_sc as plsc

N = 10000
E = 320000
D = 128
NC, NS, L = 2, 16, 16
NW = NC * NS
EPT = 10240
CH = 128
NCH = EPT // CH
GRP = 8
NG = NCH // GRP
E_PAD = NW * EPT
N_ACC = 10112
STRIPE = N_ACC // NS
BLK = 632


def _sc_agg_body(h, src, dst, zeros_a, part_out,
                 src_v, dst_v, rows_v, acc_s, gsem):
    c = lax.axis_index("c")
    s = lax.axis_index("s")
    wid = c * NS + s
    base = s * STRIPE

    pltpu.sync_copy(zeros_a, acc_s.at[pl.ds(base, STRIPE)])

    pltpu.sync_copy(src.at[wid, pl.ds(0, GRP)], src_v.at[0])
    pltpu.sync_copy(dst.at[wid, pl.ds(0, GRP)], dst_v.at[0])

    plsc.subcore_barrier()

    pltpu.async_copy(h.at[src_v.at[0, 0]], rows_v.at[0], gsem)

    def _group(g, _):
        gm = lax.rem(g, 2)
        gp = lax.rem(g + 1, 2)

        @pl.when(g + 1 < NG)
        def _():
            pltpu.sync_copy(src.at[wid, pl.ds((g + 1) * GRP, GRP)],
                            src_v.at[gp])
            pltpu.sync_copy(dst.at[wid, pl.ds((g + 1) * GRP, GRP)],
                            dst_v.at[gp])

        for k in range(GRP):
            j = g * GRP + k
            pltpu.make_async_copy(
                h.at[src_v.at[gm, k]], rows_v.at[k % 2], gsem).wait()

            @pl.when(j + 1 < NCH)
            def _():
                if k + 1 < GRP:
                    nidx = src_v.at[gm, k + 1]
                else:
                    nidx = src_v.at[gp, 0]
                pltpu.async_copy(h.at[nidx], rows_v.at[(k + 1) % 2], gsem)

            pltpu.sync_copy(rows_v.at[k % 2], acc_s.at[dst_v.at[gm, k]],
                            add=True)
        return 0
    lax.fori_loop(0, NG, _group, 0)

    plsc.subcore_barrier()

    pltpu.sync_copy(acc_s.at[pl.ds(base, STRIPE)],
                    part_out.at[c, pl.ds(base, STRIPE)])


def _sc_deg_body(dst, zeros_b, ones_in, deg_out,
                 dst_v, ones_v, dega_s, _sem):
    c = lax.axis_index("c")
    s = lax.axis_index("s")
    wid = c * NS + s
    base = s * STRIPE

    pltpu.sync_copy(zeros_b, dega_s.at[pl.ds(base, STRIPE)])
    pltpu.sync_copy(ones_in, ones_v)
    pltpu.sync_copy(dst.at[wid], dst_v)

    plsc.subcore_barrier()

    def _chunk(j, _):
        pltpu.sync_copy(ones_v, dega_s.at[dst_v.at[j]], add=True)
        return 0
    lax.fori_loop(0, NCH, _chunk, 0)

    plsc.subcore_barrier()

    pltpu.sync_copy(dega_s.at[pl.ds(base, STRIPE)],
                    deg_out.at[c, pl.ds(base, STRIPE)])


def _make_sc_agg():
    mesh = plsc.VectorSubcoreMesh(core_axis_name="c", subcore_axis_name="s")
    return pl.kernel(
        _sc_agg_body,
        out_type=jax.ShapeDtypeStruct((NC, N_ACC, D), jnp.float32),
        mesh=mesh,
        scratch_types=(
            pltpu.VMEM((2, GRP, CH), jnp.int32),
            pltpu.VMEM((2, GRP, CH), jnp.int32),
            pltpu.VMEM((2, CH, D), jnp.float32),
            pltpu.VMEM_SHARED((N_ACC, D), jnp.float32),
            pltpu.SemaphoreType.DMA,
        ),
    )


def _make_sc_deg():
    mesh = plsc.VectorSubcoreMesh(core_axis_name="c", subcore_axis_name="s")
    return pl.kernel(
        _sc_deg_body,
        out_type=jax.ShapeDtypeStruct((NC, N_ACC, D), jnp.float32),
        mesh=mesh,
        scratch_types=(
            pltpu.VMEM((NCH, CH), jnp.int32),
            pltpu.VMEM((CH, D), jnp.float32),
            pltpu.VMEM_SHARED((N_ACC, D), jnp.float32),
            pltpu.SemaphoreType.DMA,
        ),
    )


def _dense_body(relu, h_ref, p_ref, d0_ref, d1_ref, ws_ref, wn_ref, b_ref, o_ref):
    deg = d0_ref[...] + d1_ref[...]
    recip = 1.0 / jnp.maximum(deg, 1.0)
    ssum = p_ref[0] + p_ref[1]
    hn = ssum * recip
    z = (jnp.dot(h_ref[...], ws_ref[...], preferred_element_type=jnp.float32)
         + jnp.dot(hn, wn_ref[...], preferred_element_type=jnp.float32)
         + b_ref[...])
    o_ref[...] = jnp.maximum(z, 0.0) if relu else z


def _dense(h, part, d0, d1, Ws, Wn, b, relu):
    return pl.pallas_call(
        functools.partial(_dense_body, relu),
        grid=(NS,),
        in_specs=[
            pl.BlockSpec((BLK, D), lambda i: (i, 0)),
            pl.BlockSpec((NC, BLK, D), lambda i: (0, i, 0)),
            pl.BlockSpec((BLK, 1), lambda i: (i, 0)),
            pl.BlockSpec((BLK, 1), lambda i: (i, 0)),
            pl.BlockSpec((D, D), lambda i: (0, 0)),
            pl.BlockSpec((D, D), lambda i: (0, 0)),
            pl.BlockSpec((1, D), lambda i: (0, 0)),
        ],
        out_specs=pl.BlockSpec((BLK, D), lambda i: (i, 0)),
        out_shape=jax.ShapeDtypeStruct((N, D), jnp.float32),
    )(h, part, d0, d1, Ws, Wn, b)


_sc_agg = _make_sc_agg()
_sc_deg = _make_sc_deg()


def kernel(x, edge_index, W_self1, W_neigh1, b1, W_self2, W_neigh2, b2):
    src = edge_index[0]
    dst = edge_index[1]
    src_p = jnp.concatenate(
        [src, jnp.zeros((E_PAD - E,), jnp.int32)]).reshape(NW, NCH, CH)
    dst_p = jnp.concatenate(
        [dst, jnp.full((E_PAD - E,), N_ACC - 1, jnp.int32)]).reshape(NW, NCH, CH)

    zeros_a = jnp.zeros((STRIPE, D), jnp.float32)
    ones_c = jnp.ones((CH, D), jnp.float32)

    degp = _sc_deg(dst_p, zeros_a, ones_c)
    d0 = degp[0, :, 0:1]
    d1 = degp[1, :, 0:1]
    b1r = b1.reshape(1, D)
    b2r = b2.reshape(1, D)

    part1 = _sc_agg(x, src_p, dst_p, zeros_a)
    h1 = _dense(x, part1, d0, d1, W_self1, W_neigh1, b1r, relu=True)
    part2 = _sc_agg(h1, src_p, dst_p, zeros_a)
    out = _dense(h1, part2, d0, d1, W_self2, W_neigh2, b2r, relu=False)
    return out


# speedup vs baseline: 3.9862x; 1.0932x over previous
import functools

import jax
import jax.numpy as jnp
from jax import lax
from jax.experimental import pallas as pl
from jax.experimental.pallas import tpu as pltpu
from jax.experimental.pallas import tpu_sc as plsc

N = 10000
E = 320000
D = 128
NC, NS, L = 2, 16, 16
NW = NC * NS
EPT = 10240
CH = 128
NCH = EPT // CH
GRP = 8
NG = NCH // GRP
E_PAD = NW * EPT
NCHT = E_PAD // CH
NGRP_TOT = NCHT // GRP
CORE_FAST = 0
CH_FAST = 120
CH_SLOW = 40
N_ACC = 10112
STRIPE = N_ACC // NS
BLK = 632


def _sc_agg_body(h, src, dst, zeros_a, part_out,
                 src_v, dst_v, rows_v, acc_s, gsem):
    c = lax.axis_index("c")
    s = lax.axis_index("s")
    base = s * STRIPE

    cnt = jnp.where(c == CORE_FAST, CH_FAST, CH_SLOW)
    grp0 = (jnp.where(c == CORE_FAST, 0, NS * CH_FAST) + s * cnt) // GRP
    ngrp = cnt // GRP

    pltpu.sync_copy(zeros_a, acc_s.at[pl.ds(base, STRIPE)])

    pltpu.sync_copy(src.at[grp0], src_v.at[0])
    pltpu.sync_copy(dst.at[grp0], dst_v.at[0])

    plsc.subcore_barrier()

    pltpu.async_copy(h.at[src_v.at[0, 0]], rows_v.at[0], gsem)

    def _group(g, _):
        gm = lax.rem(g, 2)
        gp = lax.rem(g + 1, 2)

        @pl.when(g + 1 < ngrp)
        def _():
            pltpu.sync_copy(src.at[grp0 + g + 1], src_v.at[gp])
            pltpu.sync_copy(dst.at[grp0 + g + 1], dst_v.at[gp])

        for k in range(GRP):
            j = g * GRP + k
            pltpu.make_async_copy(
                h.at[src_v.at[gm, k]], rows_v.at[k % 2], gsem).wait()

            @pl.when(j + 1 < cnt)
            def _():
                if k + 1 < GRP:
                    nidx = src_v.at[gm, k + 1]
                else:
                    nidx = src_v.at[gp, 0]
                pltpu.async_copy(h.at[nidx], rows_v.at[(k + 1) % 2], gsem)

            pltpu.sync_copy(rows_v.at[k % 2], acc_s.at[dst_v.at[gm, k]],
                            add=True)
        return 0
    lax.fori_loop(0, ngrp, _group, 0)

    plsc.subcore_barrier()

    pltpu.sync_copy(acc_s.at[pl.ds(base, STRIPE)],
                    part_out.at[c, pl.ds(base, STRIPE)])


def _sc_deg_body(dst, zeros_b, ones_in, deg_out,
                 dst_v, ones_v, dega_s, _sem):
    c = lax.axis_index("c")
    s = lax.axis_index("s")
    wid = c * NS + s
    base = s * STRIPE

    pltpu.sync_copy(zeros_b, dega_s.at[pl.ds(base, STRIPE)])
    pltpu.sync_copy(ones_in, ones_v)
    pltpu.sync_copy(dst.at[wid], dst_v)

    plsc.subcore_barrier()

    def _chunk(j, _):
        pltpu.sync_copy(ones_v, dega_s.at[dst_v.at[j]], add=True)
        return 0
    lax.fori_loop(0, NCH, _chunk, 0)

    plsc.subcore_barrier()

    pltpu.sync_copy(dega_s.at[pl.ds(base, STRIPE)],
                    deg_out.at[c, pl.ds(base, STRIPE)])


def _make_sc_agg():
    mesh = plsc.VectorSubcoreMesh(core_axis_name="c", subcore_axis_name="s")
    return pl.kernel(
        _sc_agg_body,
        out_type=jax.ShapeDtypeStruct((NC, N_ACC, D), jnp.float32),
        mesh=mesh,
        scratch_types=(
            pltpu.VMEM((2, GRP, CH), jnp.int32),
            pltpu.VMEM((2, GRP, CH), jnp.int32),
            pltpu.VMEM((2, CH, D), jnp.float32),
            pltpu.VMEM_SHARED((N_ACC, D), jnp.float32),
            pltpu.SemaphoreType.DMA,
        ),
    )


def _make_sc_deg():
    mesh = plsc.VectorSubcoreMesh(core_axis_name="c", subcore_axis_name="s")
    return pl.kernel(
        _sc_deg_body,
        out_type=jax.ShapeDtypeStruct((NC, N_ACC, D), jnp.float32),
        mesh=mesh,
        scratch_types=(
            pltpu.VMEM((NCH, CH), jnp.int32),
            pltpu.VMEM((CH, D), jnp.float32),
            pltpu.VMEM_SHARED((N_ACC, D), jnp.float32),
            pltpu.SemaphoreType.DMA,
        ),
    )


def _dense_body(relu, h_ref, p_ref, d0_ref, d1_ref, ws_ref, wn_ref, b_ref, o_ref):
    deg = d0_ref[...] + d1_ref[...]
    recip = 1.0 / jnp.maximum(deg, 1.0)
    ssum = p_ref[0] + p_ref[1]
    hn = ssum * recip
    z = (jnp.dot(h_ref[...], ws_ref[...], preferred_element_type=jnp.float32)
         + jnp.dot(hn, wn_ref[...], preferred_element_type=jnp.float32)
         + b_ref[...])
    o_ref[...] = jnp.maximum(z, 0.0) if relu else z


def _dense(h, part, d0, d1, Ws, Wn, b, relu):
    return pl.pallas_call(
        functools.partial(_dense_body, relu),
        grid=(NS,),
        in_specs=[
            pl.BlockSpec((BLK, D), lambda i: (i, 0)),
            pl.BlockSpec((NC, BLK, D), lambda i: (0, i, 0)),
            pl.BlockSpec((BLK, 1), lambda i: (i, 0)),
            pl.BlockSpec((BLK, 1), lambda i: (i, 0)),
            pl.BlockSpec((D, D), lambda i: (0, 0)),
            pl.BlockSpec((D, D), lambda i: (0, 0)),
            pl.BlockSpec((1, D), lambda i: (0, 0)),
        ],
        out_specs=pl.BlockSpec((BLK, D), lambda i: (i, 0)),
        out_shape=jax.ShapeDtypeStruct((N, D), jnp.float32),
    )(h, part, d0, d1, Ws, Wn, b)


_sc_agg = _make_sc_agg()
_sc_deg = _make_sc_deg()


def kernel(x, edge_index, W_self1, W_neigh1, b1, W_self2, W_neigh2, b2):
    src = edge_index[0]
    dst = edge_index[1]
    src_p = jnp.concatenate(
        [src, jnp.zeros((E_PAD - E,), jnp.int32)]).reshape(NW, NCH, CH)
    dst_p = jnp.concatenate(
        [dst, jnp.full((E_PAD - E,), N_ACC - 1, jnp.int32)]).reshape(NW, NCH, CH)

    zeros_a = jnp.zeros((STRIPE, D), jnp.float32)
    ones_c = jnp.ones((CH, D), jnp.float32)

    degp = _sc_deg(dst_p, zeros_a, ones_c)
    d0 = degp[0, :, 0:1]
    d1 = degp[1, :, 0:1]
    b1r = b1.reshape(1, D)
    b2r = b2.reshape(1, D)

    src_g = src_p.reshape(NGRP_TOT, GRP, CH)
    dst_g = dst_p.reshape(NGRP_TOT, GRP, CH)

    part1 = _sc_agg(x, src_g, dst_g, zeros_a)
    h1 = _dense(x, part1, d0, d1, W_self1, W_neigh1, b1r, relu=True)
    part2 = _sc_agg(h1, src_g, dst_g, zeros_a)
    out = _dense(h1, part2, d0, d1, W_self2, W_neigh2, b2r, relu=False)
    return out


# speedup vs baseline: 4.0481x; 1.0155x over previous
import functools

import jax
import jax.numpy as jnp
from jax import lax
from jax.experimental import pallas as pl
from jax.experimental.pallas import tpu as pltpu
from jax.experimental.pallas import tpu_sc as plsc

N = 10000
E = 320000
D = 128
NC, NS, L = 2, 16, 16
NW = NC * NS
EPT = 10240
CH = 128
NCH = EPT // CH
GRP = 8
NG = NCH // GRP
E_PAD = NW * EPT
NCHT = E_PAD // CH
NGRP_TOT = NCHT // GRP
CORE_FAST = 1
CH_FAST = 120
CH_SLOW = 40
N_ACC = 10112
STRIPE = N_ACC // NS
BLK = 632


def _sc_agg_body(h, src, dst, zeros_a, part_out,
                 src_v, dst_v, rows_v, acc_s, gsem):
    c = lax.axis_index("c")
    s = lax.axis_index("s")
    base = s * STRIPE

    cnt = jnp.where(c == CORE_FAST, CH_FAST, CH_SLOW)
    grp0 = (jnp.where(c == CORE_FAST, 0, NS * CH_FAST) + s * cnt) // GRP
    ngrp = cnt // GRP

    pltpu.sync_copy(zeros_a, acc_s.at[pl.ds(base, STRIPE)])

    pltpu.sync_copy(src.at[grp0], src_v.at[0])
    pltpu.sync_copy(dst.at[grp0], dst_v.at[0])

    plsc.subcore_barrier()

    pltpu.async_copy(h.at[src_v.at[0, 0]], rows_v.at[0], gsem)

    def _group(g, _):
        gm = lax.rem(g, 2)
        gp = lax.rem(g + 1, 2)

        @pl.when(g + 1 < ngrp)
        def _():
            pltpu.sync_copy(src.at[grp0 + g + 1], src_v.at[gp])
            pltpu.sync_copy(dst.at[grp0 + g + 1], dst_v.at[gp])

        for k in range(GRP):
            j = g * GRP + k
            pltpu.make_async_copy(
                h.at[src_v.at[gm, k]], rows_v.at[k % 2], gsem).wait()

            @pl.when(j + 1 < cnt)
            def _():
                if k + 1 < GRP:
                    nidx = src_v.at[gm, k + 1]
                else:
                    nidx = src_v.at[gp, 0]
                pltpu.async_copy(h.at[nidx], rows_v.at[(k + 1) % 2], gsem)

            pltpu.sync_copy(rows_v.at[k % 2], acc_s.at[dst_v.at[gm, k]],
                            add=True)
        return 0
    lax.fori_loop(0, ngrp, _group, 0)

    plsc.subcore_barrier()

    pltpu.sync_copy(acc_s.at[pl.ds(base, STRIPE)],
                    part_out.at[c, pl.ds(base, STRIPE)])


def _sc_deg_body(dst, zeros_b, ones_in, deg_out,
                 dst_v, ones_v, dega_s, _sem):
    c = lax.axis_index("c")
    s = lax.axis_index("s")
    wid = c * NS + s
    base = s * STRIPE

    pltpu.sync_copy(zeros_b, dega_s.at[pl.ds(base, STRIPE)])
    pltpu.sync_copy(ones_in, ones_v)
    pltpu.sync_copy(dst.at[wid], dst_v)

    plsc.subcore_barrier()

    def _chunk(j, _):
        pltpu.sync_copy(ones_v, dega_s.at[dst_v.at[j]], add=True)
        return 0
    lax.fori_loop(0, NCH, _chunk, 0)

    plsc.subcore_barrier()

    pltpu.sync_copy(dega_s.at[pl.ds(base, STRIPE)],
                    deg_out.at[c, pl.ds(base, STRIPE)])


def _make_sc_agg():
    mesh = plsc.VectorSubcoreMesh(core_axis_name="c", subcore_axis_name="s")
    return pl.kernel(
        _sc_agg_body,
        out_type=jax.ShapeDtypeStruct((NC, N_ACC, D), jnp.float32),
        mesh=mesh,
        scratch_types=(
            pltpu.VMEM((2, GRP, CH), jnp.int32),
            pltpu.VMEM((2, GRP, CH), jnp.int32),
            pltpu.VMEM((2, CH, D), jnp.float32),
            pltpu.VMEM_SHARED((N_ACC, D), jnp.float32),
            pltpu.SemaphoreType.DMA,
        ),
    )


def _make_sc_deg():
    mesh = plsc.VectorSubcoreMesh(core_axis_name="c", subcore_axis_name="s")
    return pl.kernel(
        _sc_deg_body,
        out_type=jax.ShapeDtypeStruct((NC, N_ACC, D), jnp.float32),
        mesh=mesh,
        scratch_types=(
            pltpu.VMEM((NCH, CH), jnp.int32),
            pltpu.VMEM((CH, D), jnp.float32),
            pltpu.VMEM_SHARED((N_ACC, D), jnp.float32),
            pltpu.SemaphoreType.DMA,
        ),
    )


def _dense_body(relu, h_ref, p_ref, d0_ref, d1_ref, ws_ref, wn_ref, b_ref, o_ref):
    deg = d0_ref[...] + d1_ref[...]
    recip = 1.0 / jnp.maximum(deg, 1.0)
    ssum = p_ref[0] + p_ref[1]
    hn = ssum * recip
    z = (jnp.dot(h_ref[...], ws_ref[...], preferred_element_type=jnp.float32)
         + jnp.dot(hn, wn_ref[...], preferred_element_type=jnp.float32)
         + b_ref[...])
    o_ref[...] = jnp.maximum(z, 0.0) if relu else z


def _dense(h, part, d0, d1, Ws, Wn, b, relu):
    return pl.pallas_call(
        functools.partial(_dense_body, relu),
        grid=(NS,),
        in_specs=[
            pl.BlockSpec((BLK, D), lambda i: (i, 0)),
            pl.BlockSpec((NC, BLK, D), lambda i: (0, i, 0)),
            pl.BlockSpec((BLK, 1), lambda i: (i, 0)),
            pl.BlockSpec((BLK, 1), lambda i: (i, 0)),
            pl.BlockSpec((D, D), lambda i: (0, 0)),
            pl.BlockSpec((D, D), lambda i: (0, 0)),
            pl.BlockSpec((1, D), lambda i: (0, 0)),
        ],
        out_specs=pl.BlockSpec((BLK, D), lambda i: (i, 0)),
        out_shape=jax.ShapeDtypeStruct((N, D), jnp.float32),
    )(h, part, d0, d1, Ws, Wn, b)


_sc_agg = _make_sc_agg()
_sc_deg = _make_sc_deg()


def kernel(x, edge_index, W_self1, W_neigh1, b1, W_self2, W_neigh2, b2):
    src = edge_index[0]
    dst = edge_index[1]
    src_p = jnp.concatenate(
        [src, jnp.zeros((E_PAD - E,), jnp.int32)]).reshape(NW, NCH, CH)
    dst_p = jnp.concatenate(
        [dst, jnp.full((E_PAD - E,), N_ACC - 1, jnp.int32)]).reshape(NW, NCH, CH)

    zeros_a = jnp.zeros((STRIPE, D), jnp.float32)
    ones_c = jnp.ones((CH, D), jnp.float32)

    degp = _sc_deg(dst_p, zeros_a, ones_c)
    d0 = degp[0, :, 0:1]
    d1 = degp[1, :, 0:1]
    b1r = b1.reshape(1, D)
    b2r = b2.reshape(1, D)

    src_g = src_p.reshape(NGRP_TOT, GRP, CH)
    dst_g = dst_p.reshape(NGRP_TOT, GRP, CH)

    part1 = _sc_agg(x, src_g, dst_g, zeros_a)
    h1 = _dense(x, part1, d0, d1, W_self1, W_neigh1, b1r, relu=True)
    part2 = _sc_agg(h1, src_g, dst_g, zeros_a)
    out = _dense(h1, part2, d0, d1, W_self2, W_neigh2, b2r, relu=False)
    return out
